# Initial kernel scaffold; baseline (speedup 1.0000x reference)
import jax
import jax.numpy as jnp
from jax import lax
from jax.experimental import pallas as pl
from jax.experimental.pallas import tpu as pltpu
from jax.experimental.pallas import tpu_sc as plsc

H_NODES = 10000
E_RES = 19900
M_EDGES = 320000
D = 128
D_EDGE = 20
B = 200

NW = 32
EPW = M_EDGES // NW
CH = 125
NCH = EPW // CH
NSUB = 16
RPS = H_NODES // NSUB
NP = 19904
CNT_W = 16

_MESH = plsc.VectorSubcoreMesh(core_axis_name="c", subcore_axis_name="s")


def _sc_count_body(sidx, zrows, ones_h, out, sv, ones_v, acc):
    c = lax.axis_index("c")
    s = lax.axis_index("s")
    wid = s * 2 + c
    pltpu.sync_copy(sidx.at[wid], sv)
    pltpu.sync_copy(ones_h, ones_v)
    pltpu.sync_copy(zrows, acc.at[pl.ds(s * RPS, RPS)])
    plsc.subcore_barrier()

    def body(j, carry):
        pltpu.sync_copy(ones_v, acc.at[sv.at[j]], add=True)
        return carry

    lax.fori_loop(0, NCH, body, 0)
    plsc.subcore_barrier()
    pltpu.sync_copy(acc.at[pl.ds(s * RPS, RPS)], out.at[c, pl.ds(s * RPS, RPS)])


_sc_count = pl.kernel(
    _sc_count_body,
    out_type=jax.ShapeDtypeStruct((2, H_NODES, CNT_W), jnp.float32),
    mesh=_MESH,
    scratch_types=[
        pltpu.VMEM((NCH, CH), jnp.int32),
        pltpu.VMEM((CH, CNT_W), jnp.float32),
        pltpu.VMEM_SHARED((H_NODES, CNT_W), jnp.float32),
    ],
)


def _sc_gs_body(table, gidx, sidx, zrows, out, gv, sv, rows, acc):
    c = lax.axis_index("c")
    s = lax.axis_index("s")
    wid = s * 2 + c
    pltpu.sync_copy(gidx.at[wid], gv)
    pltpu.sync_copy(sidx.at[wid], sv)
    pltpu.sync_copy(zrows, acc.at[pl.ds(s * RPS, RPS)])
    plsc.subcore_barrier()

    def body(j, carry):
        pltpu.sync_copy(table.at[gv.at[j]], rows)
        pltpu.sync_copy(rows, acc.at[sv.at[j]], add=True)
        return carry

    lax.fori_loop(0, NCH, body, 0)
    plsc.subcore_barrier()
    pltpu.sync_copy(acc.at[pl.ds(s * RPS, RPS)], out.at[c, pl.ds(s * RPS, RPS)])


_sc_gs = pl.kernel(
    _sc_gs_body,
    out_type=jax.ShapeDtypeStruct((2, H_NODES, D), jnp.float32),
    mesh=_MESH,
    scratch_types=[
        pltpu.VMEM((NCH, CH), jnp.int32),
        pltpu.VMEM((NCH, CH), jnp.int32),
        pltpu.VMEM((CH, D), jnp.float32),
        pltpu.VMEM_SHARED((H_NODES, D), jnp.float32),
    ],
)


def _mean_of(s0, s1, c0, c1):
    cnt = c0[:, 0:1] + c1[:, 0:1]
    scale = 1.0 / jnp.maximum(cnt, 1.0)
    return (s0[...] + s1[...]) * scale


def _tc_edge_body(s0, s1, c0, c1, ea, w1m, w1e, b1, w2, b2, o):
    mean = _mean_of(s0, s1, c0, c1)
    h1 = jnp.dot(mean, w1m[...], preferred_element_type=jnp.float32)
    h1 = h1 + jnp.dot(ea[...], w1e[...], preferred_element_type=jnp.float32)
    h1 = jnp.maximum(h1 + b1[...], 0.0)
    o[...] = jnp.dot(h1, w2[...], preferred_element_type=jnp.float32) + b2[...]


_tc_edge = pl.pallas_call(
    _tc_edge_body,
    out_shape=jax.ShapeDtypeStruct((H_NODES, D), jnp.float32),
)


def _tc_node_body(x, s0, s1, c0, c1, wp, bp, w1, b1, w2, b2, g, bln, o):
    nm = _mean_of(s0, s1, c0, c1)
    t = jnp.maximum(jnp.dot(nm, w1[...], preferred_element_type=jnp.float32)
                    + b1[...], 0.0)
    z = (jnp.dot(x[...], wp[...], preferred_element_type=jnp.float32) + bp[...]
         + jnp.dot(t, w2[...], preferred_element_type=jnp.float32) + b2[...])
    zr = jnp.maximum(z, 0.0)
    mu = jnp.mean(zr, axis=-1, keepdims=True)
    var = jnp.mean((zr - mu) ** 2, axis=-1, keepdims=True)
    o[...] = (zr - mu) * lax.rsqrt(var + 1e-5) * g[...] + bln[...]


_tc_node = pl.pallas_call(
    _tc_node_body,
    out_shape=jax.ShapeDtypeStruct((H_NODES, D), jnp.float32),
)


def _tc_arma_body(s0, s1, c0, c1, v, ai, r0w, r1w, ww, ab0, ab1, g, bln, o):
    r10 = _mean_of(s0, s1, c0, c1)
    r = jnp.concatenate([r10, jnp.zeros((NP - H_NODES, D), jnp.float32)], 0)
    vv = v[...]
    z1 = jnp.zeros((1, 1), jnp.float32)
    vm1 = jnp.concatenate([z1, vv[:-1]], 0)
    deg = vv + vm1
    dinv = jnp.where(deg > 0, lax.rsqrt(jnp.maximum(deg, 1e-30)), 0.0)
    dinv_m1 = jnp.concatenate([z1, dinv[:-1]], 0)
    dinv_p1 = jnp.concatenate([dinv[1:], z1], 0)
    ca = vm1 * dinv_m1 * dinv
    cb = vv * dinv * dinv_p1

    zrow = jnp.zeros((1, D), jnp.float32)

    def prop(h):
        hm = jnp.concatenate([zrow, h[:-1]], 0)
        hp = jnp.concatenate([h[1:], zrow], 0)
        return ca * hm + cb * hp

    out0 = jnp.dot(r, ai[...], preferred_element_type=jnp.float32)
    t = jnp.maximum(prop(out0)
                    + jnp.dot(r, r0w[...], preferred_element_type=jnp.float32)
                    + ab0[...], 0.0)
    t = jnp.dot(t, ww[...], preferred_element_type=jnp.float32)
    t = jnp.maximum(prop(t)
                    + jnp.dot(r, r1w[...], preferred_element_type=jnp.float32)
                    + ab1[...], 0.0)
    mu = jnp.mean(t, axis=-1, keepdims=True)
    var = jnp.mean((t - mu) ** 2, axis=-1, keepdims=True)
    o[...] = (t - mu) * lax.rsqrt(var + 1e-5) * g[...] + bln[...]


_tc_arma = pl.pallas_call(
    _tc_arma_body,
    out_shape=jax.ShapeDtypeStruct((NP, D), jnp.float32),
)


def _tc_pool_body(o2, onehot, m1, b1, m2, b2, m3, b3, m4, b4, o):
    gp = jnp.dot(onehot[...], o2[...], preferred_element_type=jnp.float32)
    z = jnp.maximum(jnp.dot(gp, m1[...], preferred_element_type=jnp.float32)
                    + b1[...], 0.0)
    z = jnp.maximum(jnp.dot(z, m2[...], preferred_element_type=jnp.float32)
                    + b2[...], 0.0)
    z = jnp.maximum(jnp.dot(z, m3[...], preferred_element_type=jnp.float32)
                    + b3[...], 0.0)
    o[...] = jnp.dot(z, m4[...], preferred_element_type=jnp.float32) + b4[...]


_tc_pool = pl.pallas_call(
    _tc_pool_body,
    out_shape=jax.ShapeDtypeStruct((B, D), jnp.float32),
)


def kernel(x_h, h2_edge_index, h2_edge_attr, idx_batch, num_hyper2edges, params):
    p = params
    fg = h2_edge_index[0].astype(jnp.int32).reshape(NW, NCH, CH)
    res = h2_edge_index[1].astype(jnp.int32).reshape(NW, NCH, CH)
    zrows_d = jnp.zeros((RPS, D), jnp.float32)
    zrows_c = jnp.zeros((RPS, CNT_W), jnp.float32)
    ones_c = jnp.ones((CH, CNT_W), jnp.float32)

    cnt_res = _sc_count(res, zrows_c, ones_c)
    cnt_fg = _sc_count(fg, zrows_c, ones_c)
    cr0, cr1 = cnt_res[0], cnt_res[1]
    cf0, cf1 = cnt_fg[0], cnt_fg[1]

    ea = h2_edge_attr[:H_NODES]

    def b2d(lin):
        return lin['b'].reshape(1, -1)

    def hlayer(pre, x):
        ssum = _sc_gs(x, fg, res, zrows_d)
        w1 = p[pre + '_e1']['W']
        msg = _tc_edge(ssum[0], ssum[1], cr0, cr1, ea, w1[:D], w1[D:],
                       b2d(p[pre + '_e1']), p[pre + '_e2']['W'],
                       b2d(p[pre + '_e2']))
        nsum = _sc_gs(msg, res, fg, zrows_d)
        return _tc_node(x, nsum[0], nsum[1], cf0, cf1,
                        p[pre + '_proj']['W'], b2d(p[pre + '_proj']),
                        p[pre + '_n1']['W'], b2d(p[pre + '_n1']),
                        p[pre + '_n2']['W'], b2d(p[pre + '_n2']),
                        p[pre + '_norm']['g'].reshape(1, D),
                        p[pre + '_norm']['b'].reshape(1, D))

    h = hlayer('l1', x_h)
    h = hlayer('l2', h)
    rsum = _sc_gs(h, fg, res, zrows_d)

    seg = jnp.repeat(jnp.arange(B, dtype=jnp.int32), num_hyper2edges,
                     total_repeat_length=E_RES)
    vv = (seg[:-1] == seg[1:]).astype(jnp.float32)
    v = jnp.pad(vv, (0, NP - (E_RES - 1))).reshape(NP, 1)
    o2 = _tc_arma(rsum[0], rsum[1], cr0, cr1, v,
                  p['arma_init'][0], p['arma_root'][0, 0], p['arma_root'][1, 0],
                  p['arma_w'][0, 0],
                  p['arma_bias'][0, 0].reshape(1, D),
                  p['arma_bias'][1, 0].reshape(1, D),
                  p['res_norm']['g'].reshape(1, D),
                  p['res_norm']['b'].reshape(1, D))

    segp = jnp.pad(seg, (0, NP - E_RES), constant_values=B + 7)
    onehot = (jnp.arange(B, dtype=jnp.int32)[:, None] == segp[None, :]
              ).astype(jnp.float32)

    def padw(w, rows, cols):
        return jnp.pad(w, ((0, rows - w.shape[0]), (0, cols - w.shape[1])))

    def padb(bb, cols):
        return jnp.pad(bb, (0, cols - bb.shape[0])).reshape(1, cols)

    out = _tc_pool(o2, onehot,
                   p['m1']['W'], b2d(p['m1']),
                   padw(p['m2']['W'], 256, D), padb(p['m2']['b'], D),
                   padw(p['m3']['W'], D, D), padb(p['m3']['b'], D),
                   padw(p['m4']['W'], D, D), padb(p['m4']['b'], D))
    return out[:, 0]


# speedup vs baseline: 1.1147x; 1.1147x over previous
import jax
import jax.numpy as jnp
from jax import lax
from jax.experimental import pallas as pl
from jax.experimental.pallas import tpu as pltpu
from jax.experimental.pallas import tpu_sc as plsc

H_NODES = 10000
E_RES = 19900
M_EDGES = 320000
D = 128
D_EDGE = 20
B = 200

NW = 32
HP = 10240
RPT = HP // NW
CH = 125
CAP = 324000
NCHT = CAP // CH
SCROWS = HP // 2
ACCR = SCROWS + 128
ZR = ACCR // 16
NP = 19904


def _sc_gs_body(table, gidx, sidx, cb, zrows, out, gv, sv, rows, cb_v, acc):
    c = lax.axis_index("c")
    s = lax.axis_index("s")
    t = c * 16 + s
    pltpu.sync_copy(cb, cb_v)
    pltpu.sync_copy(zrows, acc.at[pl.ds(s * ZR, ZR)])
    plsc.subcore_barrier()

    def body(j, carry):
        pltpu.sync_copy(gidx.at[j], gv)
        pltpu.sync_copy(sidx.at[j], sv)
        pltpu.sync_copy(table.at[gv], rows)
        pltpu.sync_copy(rows, acc.at[sv], add=True)
        return carry

    cbw = cb_v[pl.ds(t, 16)]
    lax.fori_loop(cbw[0], cbw[1], body, 0)
    plsc.subcore_barrier()
    pltpu.sync_copy(acc.at[pl.ds(s * RPT, RPT)],
                    out.at[pl.ds(c * SCROWS + s * RPT, RPT)])


_SC_BUILT = {}


def _sc_gs(*args):
    if 'gs' not in _SC_BUILT:
        mesh = plsc.VectorSubcoreMesh(core_axis_name="c", subcore_axis_name="s")
        _SC_BUILT['gs'] = pl.kernel(
            _sc_gs_body,
            out_type=jax.ShapeDtypeStruct((HP, D), jnp.float32),
            mesh=mesh,
            scratch_types=[
                pltpu.VMEM((CH,), jnp.int32),
                pltpu.VMEM((CH,), jnp.int32),
                pltpu.VMEM((CH, D), jnp.float32),
                pltpu.VMEM((48,), jnp.int32),
                pltpu.VMEM_SHARED((ACCR, D), jnp.float32),
            ],
        )
    return _SC_BUILT['gs'](*args)


def _prep_key(key):
    perm = jnp.argsort(key, stable=True)
    skey = key[perm]
    starts = jnp.searchsorted(skey, jnp.arange(33, dtype=jnp.int32) * RPT
                              ).astype(jnp.int32)
    cnt = starts[1:] - starts[:-1]
    padded = ((cnt + (CH - 1)) // CH) * CH
    pstart = jnp.concatenate([jnp.zeros((1,), jnp.int32),
                              jnp.cumsum(padded).astype(jnp.int32)])
    owner = skey // RPT
    spos = jnp.arange(M_EDGES, dtype=jnp.int32)
    pos = pstart[owner] + (spos - starts[owner])
    slot = jnp.arange(CAP, dtype=jnp.int32)
    tt = jnp.clip(jnp.searchsorted(pstart, slot, side='right').astype(jnp.int32)
                  - 1, 0, 31)
    trash_local = SCROWS + (tt % 16) * 8
    sidx_pad = trash_local.at[pos].set(skey - (owner // 16) * SCROWS)
    cb = (pstart // CH).astype(jnp.int32)
    cb = jnp.pad(cb, (0, 48 - 33))
    return perm, pos, sidx_pad.reshape(NCHT, CH), cb


def _prep_gather(gkey, perm, pos):
    gidx_pad = jnp.zeros((CAP,), jnp.int32).at[pos].set(gkey[perm])
    return gidx_pad.reshape(NCHT, CH)


BS_H = 2560
BS_R = 2488


def _rows(bs, w):
    return pl.BlockSpec((bs, w), lambda i: (i, 0))


def _full(r, w):
    return pl.BlockSpec((r, w), lambda i: (0, 0))


def _mean_of(s, cnt):
    return s[...] / jnp.maximum(cnt[:, 0:1], 1.0)


def _tc_edge_body(s, cnt, ea, w1, b1, w2, b2, o):
    mean = _mean_of(s, cnt)
    e_in = jnp.concatenate([mean, ea[...]], axis=-1)
    h1 = jnp.maximum(jnp.dot(e_in, w1[...], preferred_element_type=jnp.float32)
                     + b1[...], 0.0)
    o[...] = jnp.dot(h1, w2[...], preferred_element_type=jnp.float32) + b2[...]


_tc_edge = pl.pallas_call(
    _tc_edge_body,
    grid=(HP // BS_H,),
    in_specs=[_rows(BS_H, D), _rows(BS_H, D), _rows(BS_H, D_EDGE),
              _full(D + D_EDGE, D), _full(1, D), _full(D, D), _full(1, D)],
    out_specs=_rows(BS_H, D),
    out_shape=jax.ShapeDtypeStruct((HP, D), jnp.float32),
)


def _tc_node_body(x, s, cnt, wp, bp, w1, b1, w2, b2, g, bln, o):
    nm = _mean_of(s, cnt)
    t = jnp.maximum(jnp.dot(nm, w1[...], preferred_element_type=jnp.float32)
                    + b1[...], 0.0)
    z = ((jnp.dot(x[...], wp[...], preferred_element_type=jnp.float32)
          + bp[...])
         + (jnp.dot(t, w2[...], preferred_element_type=jnp.float32) + b2[...]))
    zr = jnp.maximum(z, 0.0)
    mu = jnp.mean(zr, axis=-1, keepdims=True)
    var = jnp.mean((zr - mu) ** 2, axis=-1, keepdims=True)
    o[...] = (zr - mu) / jnp.sqrt(var + 1e-5) * g[...] + bln[...]


_tc_node = pl.pallas_call(
    _tc_node_body,
    grid=(HP // BS_H,),
    in_specs=[_rows(BS_H, D), _rows(BS_H, D), _rows(BS_H, D),
              _full(D, D), _full(1, D), _full(D, D), _full(1, D),
              _full(D, D), _full(1, D), _full(1, D), _full(1, D)],
    out_specs=_rows(BS_H, D),
    out_shape=jax.ShapeDtypeStruct((HP, D), jnp.float32),
)


def _tc_r_body(s, cnt, o):
    i = pl.program_id(0)
    rows = jnp.arange(BS_R, dtype=jnp.int32)[:, None] + i * BS_R
    mean = _mean_of(s, cnt)
    o[...] = jnp.where(rows < H_NODES, mean, 0.0)


_tc_r = pl.pallas_call(
    _tc_r_body,
    grid=(NP // BS_R,),
    in_specs=[_rows(BS_R, D), _rows(BS_R, D)],
    out_specs=_rows(BS_R, D),
    out_shape=jax.ShapeDtypeStruct((NP, D), jnp.float32),
)


def _tc_mm_body(x, w, o):
    o[...] = jnp.dot(x[...], w[...], preferred_element_type=jnp.float32)


_tc_mm = pl.pallas_call(
    _tc_mm_body,
    grid=(NP // BS_R,),
    in_specs=[_rows(BS_R, D), _full(D, D)],
    out_specs=_rows(BS_R, D),
    out_shape=jax.ShapeDtypeStruct((NP, D), jnp.float32),
)


def _coeffs(v):
    vv = v[...]
    z1 = jnp.zeros((1, 1), jnp.float32)
    vm1 = jnp.concatenate([z1, vv[:-1]], 0)
    deg = vv + vm1
    dinv = jnp.where(deg > 0, lax.rsqrt(jnp.maximum(deg, 1e-30)), 0.0)
    dinv_m1 = jnp.concatenate([z1, dinv[:-1]], 0)
    dinv_p1 = jnp.concatenate([dinv[1:], z1], 0)
    ca = vm1 * dinv_m1 * dinv
    cb = vv * dinv * dinv_p1
    return ca, cb


def _tc_prop_body(x, v, o):
    ca, cb = _coeffs(v)
    zrow = jnp.zeros((1, D), jnp.float32)
    xm = jnp.concatenate([zrow, x[:-1]], 0)
    xp = jnp.concatenate([x[1:], zrow], 0)
    o[...] = ca * xm + cb * xp


_tc_prop = pl.pallas_call(
    _tc_prop_body,
    out_shape=jax.ShapeDtypeStruct((NP, D), jnp.float32),
)


def _tc_t_body(q, r, r0w, ab0, o):
    o[...] = jnp.maximum(
        (q[...] + jnp.dot(r[...], r0w[...], preferred_element_type=jnp.float32))
        + ab0[...], 0.0)


_tc_t = pl.pallas_call(
    _tc_t_body,
    grid=(NP // BS_R,),
    in_specs=[_rows(BS_R, D), _rows(BS_R, D), _full(D, D), _full(1, D)],
    out_specs=_rows(BS_R, D),
    out_shape=jax.ShapeDtypeStruct((NP, D), jnp.float32),
)


def _tc_o2_body(q, r, r1w, ab1, g, bln, o):
    t = jnp.maximum(
        (q[...] + jnp.dot(r[...], r1w[...], preferred_element_type=jnp.float32))
        + ab1[...], 0.0)
    mu = jnp.mean(t, axis=-1, keepdims=True)
    var = jnp.mean((t - mu) ** 2, axis=-1, keepdims=True)
    o[...] = (t - mu) / jnp.sqrt(var + 1e-5) * g[...] + bln[...]


_tc_o2 = pl.pallas_call(
    _tc_o2_body,
    grid=(NP // BS_R,),
    in_specs=[_rows(BS_R, D), _rows(BS_R, D), _full(D, D), _full(1, D),
              _full(1, D), _full(1, D)],
    out_specs=_rows(BS_R, D),
    out_shape=jax.ShapeDtypeStruct((NP, D), jnp.float32),
)


def _tc_pool_body(o2, onehot, m1, b1, m2, b2, m3, b3, m4, b4, o):
    gp = jnp.dot(onehot[...], o2[...], preferred_element_type=jnp.float32,
                 precision=jax.lax.Precision.HIGHEST)
    z = jnp.maximum(jnp.dot(gp, m1[...], preferred_element_type=jnp.float32)
                    + b1[...], 0.0)
    z = jnp.maximum(jnp.dot(z, m2[...], preferred_element_type=jnp.float32)
                    + b2[...], 0.0)
    z = jnp.maximum(jnp.dot(z, m3[...], preferred_element_type=jnp.float32)
                    + b3[...], 0.0)
    o[...] = jnp.dot(z, m4[...], preferred_element_type=jnp.float32) + b4[...]


_tc_pool = pl.pallas_call(
    _tc_pool_body,
    out_shape=jax.ShapeDtypeStruct((B, D), jnp.float32),
)


def kernel(x_h, h2_edge_index, h2_edge_attr, idx_batch, num_hyper2edges, params):
    p = params
    fg = h2_edge_index[0].astype(jnp.int32)
    res = h2_edge_index[1].astype(jnp.int32)

    perm_r, pos_r, sidxA, cbA = _prep_key(res)
    gidxA = _prep_gather(fg, perm_r, pos_r)
    perm_f, pos_f, sidxB, cbB = _prep_key(fg)
    gidxB = _prep_gather(res, perm_f, pos_f)

    zrows = jnp.zeros((ZR, D), jnp.float32)
    ones_tab = jnp.ones((HP, D), jnp.float32)

    cnt_res = _sc_gs(ones_tab, gidxA, sidxA, cbA, zrows)
    cnt_fg = _sc_gs(ones_tab, gidxB, sidxB, cbB, zrows)

    ea = jnp.pad(h2_edge_attr[:H_NODES], ((0, HP - H_NODES), (0, 0)))
    x_hp = jnp.pad(x_h, ((0, HP - H_NODES), (0, 0)))

    def b2d(lin):
        return lin['b'].reshape(1, -1)

    def hlayer(pre, x):
        ssum = _sc_gs(x, gidxA, sidxA, cbA, zrows)
        msg = _tc_edge(ssum, cnt_res, ea, p[pre + '_e1']['W'],
                       b2d(p[pre + '_e1']), p[pre + '_e2']['W'],
                       b2d(p[pre + '_e2']))
        nsum = _sc_gs(msg, gidxB, sidxB, cbB, zrows)
        return _tc_node(x, nsum, cnt_fg,
                        p[pre + '_proj']['W'], b2d(p[pre + '_proj']),
                        p[pre + '_n1']['W'], b2d(p[pre + '_n1']),
                        p[pre + '_n2']['W'], b2d(p[pre + '_n2']),
                        p[pre + '_norm']['g'].reshape(1, D),
                        p[pre + '_norm']['b'].reshape(1, D))

    h = hlayer('l1', x_hp)
    h = hlayer('l2', h)
    rsum = _sc_gs(h, gidxA, sidxA, cbA, zrows)

    seg = jnp.repeat(jnp.arange(B, dtype=jnp.int32), num_hyper2edges,
                     total_repeat_length=E_RES)
    vv = (seg[:-1] == seg[1:]).astype(jnp.float32)
    v = jnp.pad(vv, (0, NP - (E_RES - 1))).reshape(NP, 1)
    rsump = jnp.pad(rsum, ((0, NP - HP), (0, 0)))
    cntp = jnp.pad(cnt_res, ((0, NP - HP), (0, 0)))
    r = _tc_r(rsump, cntp)
    mm1 = _tc_mm(r, p['arma_init'][0])
    q1 = _tc_prop(mm1, v)
    t1 = _tc_t(q1, r, p['arma_root'][0, 0], p['arma_bias'][0, 0].reshape(1, D))
    mm2 = _tc_mm(t1, p['arma_w'][0, 0])
    q2 = _tc_prop(mm2, v)
    o2 = _tc_o2(q2, r, p['arma_root'][1, 0], p['arma_bias'][1, 0].reshape(1, D),
                p['res_norm']['g'].reshape(1, D),
                p['res_norm']['b'].reshape(1, D))

    segp = jnp.pad(seg, (0, NP - E_RES), constant_values=B + 7)
    onehot = (jnp.arange(B, dtype=jnp.int32)[:, None] == segp[None, :]
              ).astype(jnp.float32)

    def padw(w, rows, cols):
        return jnp.pad(w, ((0, rows - w.shape[0]), (0, cols - w.shape[1])))

    def padb(bb, cols):
        return jnp.pad(bb, (0, cols - bb.shape[0])).reshape(1, cols)

    out = _tc_pool(o2, onehot,
                   p['m1']['W'], b2d(p['m1']),
                   padw(p['m2']['W'], 256, D), padb(p['m2']['b'], D),
                   padw(p['m3']['W'], D, D), padb(p['m3']['b'], D),
                   padw(p['m4']['W'], D, D), padb(p['m4']['b'], D))
    return out[:, 0]
